# baseline (speedup 1.0000x reference)
import functools

import jax
import jax.numpy as jnp
from jax import lax
from jax.experimental import pallas as pl
from jax.experimental.pallas import tpu as pltpu
from jax.experimental.pallas import tpu_sc as plsc

NU = 5000
NI = 5000
NN = NU + NI
D = 128
NLAYERS = 3
NC, NS, LANES = 2, 16, 16
NW = NC * NS
CK = 64
CKE = 128
CPD = D // LANES


def _mask_body(tbl_ref, sz_ref, out_ref):
    it = lax.broadcasted_iota(jnp.int32, out_ref.shape, 1)
    out_ref[...] = tbl_ref[...] * (it < sz_ref[...]).astype(jnp.float32)


def _masked(tbl, sizes):
    return pl.pallas_call(
        _mask_body,
        out_shape=jax.ShapeDtypeStruct(tbl.shape, jnp.float32),
    )(tbl, sizes.astype(jnp.int32).reshape(-1, 1))


def _combine_body(scale):
    def body(p_ref, s_ref, y_ref, so_ref):
        y = p_ref[0] + p_ref[1]
        y_ref[...] = y
        so_ref[...] = (s_ref[...] + y) * scale
    return body


def _combine(part, s, scale):
    rb = NN // 10
    return pl.pallas_call(
        _combine_body(scale),
        grid=(10,),
        in_specs=[
            pl.BlockSpec((2, rb, D), lambda i: (0, i, 0)),
            pl.BlockSpec((rb, D), lambda i: (i, 0)),
        ],
        out_specs=[
            pl.BlockSpec((rb, D), lambda i: (i, 0)),
            pl.BlockSpec((rb, D), lambda i: (i, 0)),
        ],
        out_shape=[
            jax.ShapeDtypeStruct((NN, D), jnp.float32),
            jax.ShapeDtypeStruct((NN, D), jnp.float32),
        ],
    )(part, s)


SEC = 5
NBUF = 5
LEAD = 4


def _sc_layer(x, src4, dst4, w4):
    nsec = src4.shape[0] // NW
    nct = nsec * SEC
    nround = nsec // 2
    assert nsec % 2 == 0
    mesh = plsc.VectorSubcoreMesh(core_axis_name="c", subcore_axis_name="s")

    @functools.partial(
        pl.kernel,
        out_type=jax.ShapeDtypeStruct((NC, NN, D), jnp.float32),
        mesh=mesh,
        scratch_types=[
            [pltpu.VMEM((SEC, CK), jnp.int32) for _ in range(2)],
            [pltpu.VMEM((SEC, CK), jnp.int32) for _ in range(2)],
            [pltpu.VMEM((SEC, CK), jnp.float32) for _ in range(2)],
            [pltpu.VMEM((CK, D), jnp.float32) for _ in range(NBUF)],
            pltpu.VMEM_SHARED((NN, D), jnp.float32),
            [pltpu.SemaphoreType.DMA for _ in range(2)],
            [pltpu.SemaphoreType.DMA for _ in range(NBUF)],
            [pltpu.SemaphoreType.DMA for _ in range(NBUF)],
        ],
    )
    def k(x_hbm, src_hbm, dst_hbm, w_hbm, part_hbm,
          src_s, dst_s, w_s, bufs, acc, isem, gsem, ssem):
        cid = lax.axis_index("c")
        sid = lax.axis_index("s")
        wid = sid * NC + cid

        zbuf = bufs[0]

        def zrow(r, _):
            for c in range(CPD):
                zbuf[r, pl.ds(c * LANES, LANES)] = jnp.zeros((LANES,), jnp.float32)
            return 0
        lax.fori_loop(0, CK, zrow, 0)

        rpt = 624
        nfull = rpt // CK
        rem = rpt - nfull * CK
        tail_base = NS * rpt
        tail = NN - tail_base
        base = sid * rpt
        for t in range(nfull):
            pltpu.sync_copy(zbuf, acc.at[pl.ds(base + t * CK, CK)])
        if rem:
            pltpu.sync_copy(zbuf.at[pl.ds(0, rem)],
                            acc.at[pl.ds(base + nfull * CK, rem)])

        @pl.when(sid == NS - 1)
        def _():
            pltpu.sync_copy(zbuf.at[pl.ds(0, tail)],
                            acc.at[pl.ds(tail_base, tail)])
        plsc.subcore_barrier()

        def load_sec(s, p):
            b = wid * nsec + s
            pltpu.async_copy(src_hbm.at[b], src_s[p], isem[p])
            pltpu.async_copy(dst_hbm.at[b], dst_s[p], isem[p])
            pltpu.async_copy(w_hbm.at[b], w_s[p], isem[p])

        def wait_sec(p):
            pltpu.make_async_copy(src_hbm.at[0], src_s[p], isem[p]).wait()
            pltpu.make_async_copy(dst_hbm.at[0], dst_s[p], isem[p]).wait()
            pltpu.make_async_copy(w_hbm.at[0], w_s[p], isem[p]).wait()

        def start_gather(p, jj, b):
            pltpu.async_copy(x_hbm.at[src_s[p].at[jj]], bufs[b], gsem[b])

        load_sec(0, 0)
        wait_sec(0)
        load_sec(1, 1)
        for j0 in range(LEAD):
            start_gather(0, j0, j0)

        def rnd(t, _):
            for p in range(2):
                s = 2 * t + p

                @pl.when(s + 1 < nsec)
                def _():
                    wait_sec(1 - p)
                for jj in range(SEC):
                    c = 2 * SEC * t + SEC * p + jj
                    b = jj
                    b2 = (jj + LEAD) % NBUF
                    pltpu.make_async_copy(x_hbm.at[src_s[p].at[jj]],
                                          bufs[b], gsem[b]).wait()

                    @pl.when(c >= 1)
                    def _():
                        pltpu.make_async_copy(bufs[b2],
                                              acc.at[dst_s[p].at[jj]],
                                              ssem[b2]).wait()

                    @pl.when(c + LEAD < nct)
                    def _():
                        if jj == 0:
                            start_gather(p, LEAD, b2)
                        else:
                            start_gather(1 - p, jj - 1, b2)

                    def grp_body(g, _, _b=b, _p=p, _jj=jj):
                        wgrp = w_s[_p][_jj, pl.ds(g * LANES, LANES)]
                        for lane in range(LANES):
                            wv = wgrp[lane]
                            r = g * LANES + lane
                            for cc in range(CPD):
                                bufs[_b][r, pl.ds(cc * LANES, LANES)] = (
                                    bufs[_b][r, pl.ds(cc * LANES, LANES)] * wv)
                        return 0
                    lax.fori_loop(0, CK // LANES, grp_body, 0)
                    pltpu.async_copy(bufs[b], acc.at[dst_s[p].at[jj]],
                                     ssem[b], add=True)

                @pl.when(s + 2 < nsec)
                def _():
                    load_sec(s + 2, p)
            return 0
        lax.fori_loop(0, nround, rnd, 0)

        bl = (nct - 1) % NBUF
        pl_last = (nsec - 1) % 2
        pltpu.make_async_copy(bufs[bl], acc.at[dst_s[pl_last].at[SEC - 1]],
                              ssem[bl]).wait()
        plsc.subcore_barrier()

        for t in range(nfull):
            pltpu.sync_copy(acc.at[pl.ds(base + t * CK, CK)],
                            part_hbm.at[cid, pl.ds(base + t * CK, CK)])
        if rem:
            pltpu.sync_copy(acc.at[pl.ds(base + nfull * CK, rem)],
                            part_hbm.at[cid, pl.ds(base + nfull * CK, rem)])

        @pl.when(sid == NS - 1)
        def _():
            pltpu.sync_copy(acc.at[pl.ds(tail_base, tail)],
                            part_hbm.at[cid, pl.ds(tail_base, tail)])

    return k(x, src4, dst4, w4)


def _sc_gather(light_tab, ego_tab, cat3):
    npc = cat3.shape[1]
    rows_pw = npc * CKE
    nrows = NW * rows_pw
    mesh = plsc.VectorSubcoreMesh(core_axis_name="c", subcore_axis_name="s")

    @functools.partial(
        pl.kernel,
        out_type=(
            jax.ShapeDtypeStruct((nrows, D), jnp.float32),
            jax.ShapeDtypeStruct((nrows, D), jnp.float32),
        ),
        mesh=mesh,
        scratch_types=[
            pltpu.VMEM((npc, CKE), jnp.int32),
            pltpu.VMEM((CKE, D), jnp.float32),
            pltpu.VMEM((CKE, D), jnp.float32),
            pltpu.SemaphoreType.DMA,
            pltpu.SemaphoreType.DMA,
        ],
    )
    def k(lt_hbm, et_hbm, idx_hbm, lo_hbm, eo_hbm, idx_v, bufa, bufb, sa, sb):
        cid = lax.axis_index("c")
        sid = lax.axis_index("s")
        wid = sid * NC + cid
        pltpu.sync_copy(idx_hbm.at[wid], idx_v)

        def chunk(j, _):
            ca = pltpu.async_copy(lt_hbm.at[idx_v.at[j]], bufa, sa)
            cb = pltpu.async_copy(et_hbm.at[idx_v.at[j]], bufb, sb)
            ca.wait()
            pltpu.sync_copy(bufa, lo_hbm.at[pl.ds(wid * rows_pw + j * CKE, CKE)])
            cb.wait()
            pltpu.sync_copy(bufb, eo_hbm.at[pl.ds(wid * rows_pw + j * CKE, CKE)])
            return 0
        lax.fori_loop(0, npc, chunk, 0)

    return k(light_tab, ego_tab, cat3)


def kernel(users, pos_items, neg_items, edge_index, edge_weight,
           user_sizes, item_sizes, user_table, item_table):
    ue = _masked(user_table, user_sizes)
    ie = _masked(item_table, item_sizes)
    x0 = jnp.concatenate([ue, ie], axis=0)

    src = edge_index[0].astype(jnp.int32)
    dst = edge_index[1].astype(jnp.int32)
    w = edge_weight.astype(jnp.float32)
    e = src.shape[0]
    unit = NW * SEC * CK * 2
    epad = ((e + unit - 1) // unit) * unit
    pad = epad - e
    nws = epad // (SEC * CK)
    src4 = jnp.pad(src, (0, pad)).reshape(nws, SEC, CK)
    dst4 = jnp.pad(dst, (0, pad)).reshape(nws, SEC, CK)
    w4 = jnp.pad(w, (0, pad)).reshape(nws, SEC, CK)

    x = x0
    s = x0
    for l in range(NLAYERS):
        part = _sc_layer(x, src4, dst4, w4)
        x, s = _combine(part, s, 0.25 if l == NLAYERS - 1 else 1.0)

    b = users.shape[0]
    cat = jnp.concatenate([
        users.astype(jnp.int32),
        pos_items.astype(jnp.int32) + NU,
        neg_items.astype(jnp.int32) + NU,
    ])
    cat3 = cat.reshape(NW, -1, CKE)
    light, ego = _sc_gather(s, x0, cat3)
    return (light[:b], light[b:2 * b], light[2 * b:],
            ego[:b], ego[b:2 * b], ego[2 * b:])


# speedup vs baseline: 1.9428x; 1.9428x over previous
import functools

import jax
import jax.numpy as jnp
from jax import lax
from jax.experimental import pallas as pl
from jax.experimental.pallas import tpu as pltpu
from jax.experimental.pallas import tpu_sc as plsc

NU = 5000
NI = 5000
NN = NU + NI
D = 128
NLAYERS = 3
NC, NS, LANES = 2, 16, 16
NW = NC * NS
CK = 112
CKE = 128
CPD = D // LANES


def _mask_body(u_ref, i_ref, us_ref, is_ref, out_ref):
    it = lax.broadcasted_iota(jnp.int32, (NU, D), 1)
    out_ref[pl.ds(0, NU), :] = (
        u_ref[...] * (it < us_ref[...]).astype(jnp.float32))
    out_ref[pl.ds(NU, NI), :] = (
        i_ref[...] * (it < is_ref[...]).astype(jnp.float32))


def _masked_concat(user_table, item_table, user_sizes, item_sizes):
    return pl.pallas_call(
        _mask_body,
        out_shape=jax.ShapeDtypeStruct((NN, D), jnp.float32),
    )(user_table, item_table,
      user_sizes.astype(jnp.int32).reshape(-1, 1),
      item_sizes.astype(jnp.int32).reshape(-1, 1))


def _combine_body(scale):
    def body(p_ref, s_ref, y_ref, so_ref):
        y = p_ref[0] + p_ref[1]
        y_ref[...] = y
        so_ref[...] = (s_ref[...] + y) * scale
    return body


def _combine(part, s, scale):
    rb = NN // 10
    return pl.pallas_call(
        _combine_body(scale),
        grid=(10,),
        in_specs=[
            pl.BlockSpec((2, rb, D), lambda i: (0, i, 0)),
            pl.BlockSpec((rb, D), lambda i: (i, 0)),
        ],
        out_specs=[
            pl.BlockSpec((rb, D), lambda i: (i, 0)),
            pl.BlockSpec((rb, D), lambda i: (i, 0)),
        ],
        out_shape=[
            jax.ShapeDtypeStruct((NN, D), jnp.float32),
            jax.ShapeDtypeStruct((NN, D), jnp.float32),
        ],
    )(part, s)


SEC = 3
NBUF = 3


def _sc_layer(x, src4, dst4, w4):
    nsec = src4.shape[0] // NW
    nct = nsec * SEC
    nround = nsec // 2
    assert nsec % 2 == 0
    mesh = plsc.VectorSubcoreMesh(core_axis_name="c", subcore_axis_name="s")

    @functools.partial(
        pl.kernel,
        out_type=jax.ShapeDtypeStruct((NC, NN, D), jnp.float32),
        mesh=mesh,
        scratch_types=[
            [pltpu.VMEM((SEC, CK), jnp.int32) for _ in range(2)],
            [pltpu.VMEM((SEC, CK), jnp.int32) for _ in range(2)],
            [pltpu.VMEM((SEC, CK), jnp.float32) for _ in range(2)],
            [pltpu.VMEM((CK, D), jnp.float32) for _ in range(NBUF)],
            pltpu.VMEM_SHARED((NN, D), jnp.float32),
            [pltpu.SemaphoreType.DMA for _ in range(2)],
            [pltpu.SemaphoreType.DMA for _ in range(NBUF)],
            [pltpu.SemaphoreType.DMA for _ in range(NBUF)],
        ],
    )
    def k(x_hbm, src_hbm, dst_hbm, w_hbm, part_hbm,
          src_s, dst_s, w_s, bufs, acc, isem, gsem, ssem):
        cid = lax.axis_index("c")
        sid = lax.axis_index("s")
        wid = sid * NC + cid

        zbuf = bufs[0]

        def zrow(r, _):
            for c in range(CPD):
                zbuf[r, pl.ds(c * LANES, LANES)] = jnp.zeros((LANES,), jnp.float32)
            return 0
        lax.fori_loop(0, CK, zrow, 0)

        rpt = 624
        nfull = rpt // CK
        rem = rpt - nfull * CK
        tail_base = NS * rpt
        tail = NN - tail_base
        base = sid * rpt
        for t in range(nfull):
            pltpu.sync_copy(zbuf, acc.at[pl.ds(base + t * CK, CK)])
        if rem:
            pltpu.sync_copy(zbuf.at[pl.ds(0, rem)],
                            acc.at[pl.ds(base + nfull * CK, rem)])

        @pl.when(sid == NS - 1)
        def _():
            pltpu.sync_copy(zbuf.at[pl.ds(0, tail)],
                            acc.at[pl.ds(tail_base, tail)])
        plsc.subcore_barrier()

        def load_sec(s, p):
            b = wid * nsec + s
            pltpu.async_copy(src_hbm.at[b], src_s[p], isem[p])
            pltpu.async_copy(dst_hbm.at[b], dst_s[p], isem[p])
            pltpu.async_copy(w_hbm.at[b], w_s[p], isem[p])

        def wait_sec(p):
            pltpu.make_async_copy(src_hbm.at[0], src_s[p], isem[p]).wait()
            pltpu.make_async_copy(dst_hbm.at[0], dst_s[p], isem[p]).wait()
            pltpu.make_async_copy(w_hbm.at[0], w_s[p], isem[p]).wait()

        def start_gather(p, jj, b):
            pltpu.async_copy(x_hbm.at[src_s[p].at[jj]], bufs[b], gsem[b])

        load_sec(0, 0)
        wait_sec(0)
        load_sec(1, 1)
        start_gather(0, 0, 0)
        start_gather(0, 1, 1)

        def rnd(t, _):
            for p in range(2):
                s = 2 * t + p

                @pl.when(s + 1 < nsec)
                def _():
                    wait_sec(1 - p)
                for jj in range(SEC):
                    c = 2 * SEC * t + SEC * p + jj
                    b = (SEC * p + jj) % NBUF
                    b2 = (b + 2) % NBUF
                    pltpu.make_async_copy(x_hbm.at[src_s[p].at[jj]],
                                          bufs[b], gsem[b]).wait()

                    @pl.when(c >= 1)
                    def _():
                        pltpu.make_async_copy(bufs[b2],
                                              acc.at[dst_s[p].at[jj]],
                                              ssem[b2]).wait()

                    @pl.when(c + 2 < nct)
                    def _():
                        if jj == 0:
                            start_gather(p, 2, b2)
                        else:
                            start_gather(1 - p, jj - 1, b2)

                    def grp_body(g, _, _b=b, _p=p, _jj=jj):
                        wgrp = w_s[_p][_jj, pl.ds(g * LANES, LANES)]
                        for lane in range(LANES):
                            wv = wgrp[lane]
                            r = g * LANES + lane
                            for cc in range(CPD):
                                bufs[_b][r, pl.ds(cc * LANES, LANES)] = (
                                    bufs[_b][r, pl.ds(cc * LANES, LANES)] * wv)
                        return 0
                    lax.fori_loop(0, CK // LANES, grp_body, 0)
                    pltpu.async_copy(bufs[b], acc.at[dst_s[p].at[jj]],
                                     ssem[b], add=True)

                @pl.when(s + 2 < nsec)
                def _():
                    load_sec(s + 2, p)
            return 0
        lax.fori_loop(0, nround, rnd, 0)

        bl = (nct - 1) % NBUF
        pl_last = (nsec - 1) % 2
        pltpu.make_async_copy(bufs[bl], acc.at[dst_s[pl_last].at[SEC - 1]],
                              ssem[bl]).wait()
        plsc.subcore_barrier()

        for t in range(nfull):
            pltpu.sync_copy(acc.at[pl.ds(base + t * CK, CK)],
                            part_hbm.at[cid, pl.ds(base + t * CK, CK)])
        if rem:
            pltpu.sync_copy(acc.at[pl.ds(base + nfull * CK, rem)],
                            part_hbm.at[cid, pl.ds(base + nfull * CK, rem)])

        @pl.when(sid == NS - 1)
        def _():
            pltpu.sync_copy(acc.at[pl.ds(tail_base, tail)],
                            part_hbm.at[cid, pl.ds(tail_base, tail)])

    return k(x, src4, dst4, w4)


def _sc_gather(light_tab, ego_tab, cat3):
    npc = cat3.shape[1]
    rows_pw = npc * CKE
    nrows = NW * rows_pw
    mesh = plsc.VectorSubcoreMesh(core_axis_name="c", subcore_axis_name="s")

    @functools.partial(
        pl.kernel,
        out_type=(
            jax.ShapeDtypeStruct((nrows, D), jnp.float32),
            jax.ShapeDtypeStruct((nrows, D), jnp.float32),
        ),
        mesh=mesh,
        scratch_types=[
            pltpu.VMEM((npc, CKE), jnp.int32),
            pltpu.VMEM((CKE, D), jnp.float32),
            pltpu.VMEM((CKE, D), jnp.float32),
            pltpu.SemaphoreType.DMA,
            pltpu.SemaphoreType.DMA,
        ],
    )
    def k(lt_hbm, et_hbm, idx_hbm, lo_hbm, eo_hbm, idx_v, bufa, bufb, sa, sb):
        cid = lax.axis_index("c")
        sid = lax.axis_index("s")
        wid = sid * NC + cid
        pltpu.sync_copy(idx_hbm.at[wid], idx_v)

        def chunk(j, _):
            ca = pltpu.async_copy(lt_hbm.at[idx_v.at[j]], bufa, sa)
            cb = pltpu.async_copy(et_hbm.at[idx_v.at[j]], bufb, sb)
            ca.wait()
            pltpu.sync_copy(bufa, lo_hbm.at[pl.ds(wid * rows_pw + j * CKE, CKE)])
            cb.wait()
            pltpu.sync_copy(bufb, eo_hbm.at[pl.ds(wid * rows_pw + j * CKE, CKE)])
            return 0
        lax.fori_loop(0, npc, chunk, 0)

    return k(light_tab, ego_tab, cat3)


def kernel(users, pos_items, neg_items, edge_index, edge_weight,
           user_sizes, item_sizes, user_table, item_table):
    x0 = _masked_concat(user_table, item_table, user_sizes, item_sizes)

    src = edge_index[0].astype(jnp.int32)
    dst = edge_index[1].astype(jnp.int32)
    w = edge_weight.astype(jnp.float32)
    e = src.shape[0]
    unit = NW * SEC * CK * 2
    epad = ((e + unit - 1) // unit) * unit
    pad = epad - e
    nws = epad // (SEC * CK)
    src4 = jnp.pad(src, (0, pad)).reshape(nws, SEC, CK)
    dst4 = jnp.pad(dst, (0, pad)).reshape(nws, SEC, CK)
    w4 = jnp.pad(w, (0, pad)).reshape(nws, SEC, CK)

    x = x0
    s = x0
    for l in range(NLAYERS):
        part = _sc_layer(x, src4, dst4, w4)
        x, s = _combine(part, s, 0.25 if l == NLAYERS - 1 else 1.0)

    b = users.shape[0]
    cat = jnp.concatenate([
        users.astype(jnp.int32),
        pos_items.astype(jnp.int32) + NU,
        neg_items.astype(jnp.int32) + NU,
    ])
    cat3 = cat.reshape(NW, -1, CKE)
    light, ego = _sc_gather(s, x0, cat3)
    return (light[:b], light[b:2 * b], light[2 * b:],
            ego[:b], ego[b:2 * b], ego[2 * b:])


# speedup vs baseline: 1.9885x; 1.0235x over previous
import functools

import jax
import jax.numpy as jnp
from jax import lax
from jax.experimental import pallas as pl
from jax.experimental.pallas import tpu as pltpu
from jax.experimental.pallas import tpu_sc as plsc

NU = 5000
NI = 5000
NN = NU + NI
D = 128
NLAYERS = 3
NC, NS, LANES = 2, 16, 16
NW = NC * NS
CK = 112
CKE = 128
CPD = D // LANES


def _mask_body(tbl_ref, sz_ref, out_ref):
    it = lax.broadcasted_iota(jnp.int32, out_ref.shape, 1)
    out_ref[...] = tbl_ref[...] * (it < sz_ref[...]).astype(jnp.float32)


def _masked(tbl, sizes):
    return pl.pallas_call(
        _mask_body,
        out_shape=jax.ShapeDtypeStruct(tbl.shape, jnp.float32),
    )(tbl, sizes.astype(jnp.int32).reshape(-1, 1))


def _combine_body(scale):
    def body(p_ref, s_ref, y_ref, so_ref):
        y = p_ref[0] + p_ref[1]
        y_ref[...] = y
        so_ref[...] = (s_ref[...] + y) * scale
    return body


def _combine(part, s, scale):
    rb = NN // 10
    return pl.pallas_call(
        _combine_body(scale),
        grid=(10,),
        in_specs=[
            pl.BlockSpec((2, rb, D), lambda i: (0, i, 0)),
            pl.BlockSpec((rb, D), lambda i: (i, 0)),
        ],
        out_specs=[
            pl.BlockSpec((rb, D), lambda i: (i, 0)),
            pl.BlockSpec((rb, D), lambda i: (i, 0)),
        ],
        out_shape=[
            jax.ShapeDtypeStruct((NN, D), jnp.float32),
            jax.ShapeDtypeStruct((NN, D), jnp.float32),
        ],
    )(part, s)


SEC = 3
NBUF = 3


N0 = 28


def _sc_layer(x, src4, dst4, w4):
    nsec2 = src4.shape[0] // NS
    n1 = nsec2 - N0
    assert N0 % 2 == 0 and n1 % 2 == 0
    mesh = plsc.VectorSubcoreMesh(core_axis_name="c", subcore_axis_name="s")

    @functools.partial(
        pl.kernel,
        out_type=jax.ShapeDtypeStruct((NC, NN, D), jnp.float32),
        mesh=mesh,
        scratch_types=[
            [pltpu.VMEM((SEC, CK), jnp.int32) for _ in range(2)],
            [pltpu.VMEM((SEC, CK), jnp.int32) for _ in range(2)],
            [pltpu.VMEM((SEC, CK), jnp.float32) for _ in range(2)],
            [pltpu.VMEM((CK, D), jnp.float32) for _ in range(NBUF)],
            pltpu.VMEM_SHARED((NN, D), jnp.float32),
            [pltpu.SemaphoreType.DMA for _ in range(2)],
            [pltpu.SemaphoreType.DMA for _ in range(NBUF)],
            [pltpu.SemaphoreType.DMA for _ in range(NBUF)],
        ],
    )
    def k(x_hbm, src_hbm, dst_hbm, w_hbm, part_hbm,
          src_s, dst_s, w_s, bufs, acc, isem, gsem, ssem):
        cid = lax.axis_index("c")
        sid = lax.axis_index("s")
        sec_base = sid * nsec2 + cid * N0
        nsec = jnp.where(cid == 0, N0, n1)
        nct = nsec * SEC
        nround = nsec // 2

        zbuf = bufs[0]

        def zrow(r, _):
            for c in range(CPD):
                zbuf[r, pl.ds(c * LANES, LANES)] = jnp.zeros((LANES,), jnp.float32)
            return 0
        lax.fori_loop(0, CK, zrow, 0)

        rpt = 624
        nfull = rpt // CK
        rem = rpt - nfull * CK
        tail_base = NS * rpt
        tail = NN - tail_base
        base = sid * rpt
        for t in range(nfull):
            pltpu.sync_copy(zbuf, acc.at[pl.ds(base + t * CK, CK)])
        if rem:
            pltpu.sync_copy(zbuf.at[pl.ds(0, rem)],
                            acc.at[pl.ds(base + nfull * CK, rem)])

        @pl.when(sid == NS - 1)
        def _():
            pltpu.sync_copy(zbuf.at[pl.ds(0, tail)],
                            acc.at[pl.ds(tail_base, tail)])
        plsc.subcore_barrier()

        def load_sec(s, p):
            b = sec_base + s
            pltpu.async_copy(src_hbm.at[b], src_s[p], isem[p])
            pltpu.async_copy(dst_hbm.at[b], dst_s[p], isem[p])
            pltpu.async_copy(w_hbm.at[b], w_s[p], isem[p])

        def wait_sec(p):
            pltpu.make_async_copy(src_hbm.at[0], src_s[p], isem[p]).wait()
            pltpu.make_async_copy(dst_hbm.at[0], dst_s[p], isem[p]).wait()
            pltpu.make_async_copy(w_hbm.at[0], w_s[p], isem[p]).wait()

        def start_gather(p, jj, b):
            pltpu.async_copy(x_hbm.at[src_s[p].at[jj]], bufs[b], gsem[b])

        load_sec(0, 0)
        wait_sec(0)
        load_sec(1, 1)
        start_gather(0, 0, 0)
        start_gather(0, 1, 1)

        def rnd(t, _):
            for p in range(2):
                s = 2 * t + p

                @pl.when(s + 1 < nsec)
                def _():
                    wait_sec(1 - p)
                for jj in range(SEC):
                    c = 2 * SEC * t + SEC * p + jj
                    b = (SEC * p + jj) % NBUF
                    b2 = (b + 2) % NBUF
                    pltpu.make_async_copy(x_hbm.at[src_s[p].at[jj]],
                                          bufs[b], gsem[b]).wait()

                    @pl.when(c >= 1)
                    def _():
                        pltpu.make_async_copy(bufs[b2],
                                              acc.at[dst_s[p].at[jj]],
                                              ssem[b2]).wait()

                    @pl.when(c + 2 < nct)
                    def _():
                        if jj == 0:
                            start_gather(p, 2, b2)
                        else:
                            start_gather(1 - p, jj - 1, b2)

                    def grp_body(g, _, _b=b, _p=p, _jj=jj):
                        wgrp = w_s[_p][_jj, pl.ds(g * LANES, LANES)]
                        for lane in range(LANES):
                            wv = wgrp[lane]
                            r = g * LANES + lane
                            for cc in range(CPD):
                                bufs[_b][r, pl.ds(cc * LANES, LANES)] = (
                                    bufs[_b][r, pl.ds(cc * LANES, LANES)] * wv)
                        return 0
                    lax.fori_loop(0, CK // LANES, grp_body, 0)
                    pltpu.async_copy(bufs[b], acc.at[dst_s[p].at[jj]],
                                     ssem[b], add=True)

                @pl.when(s + 2 < nsec)
                def _():
                    load_sec(s + 2, p)
            return 0
        lax.fori_loop(0, nround, rnd, 0)

        bl = SEC - 1
        pl_last = 1
        pltpu.make_async_copy(bufs[bl], acc.at[dst_s[pl_last].at[SEC - 1]],
                              ssem[bl]).wait()
        plsc.subcore_barrier()

        for t in range(nfull):
            pltpu.sync_copy(acc.at[pl.ds(base + t * CK, CK)],
                            part_hbm.at[cid, pl.ds(base + t * CK, CK)])
        if rem:
            pltpu.sync_copy(acc.at[pl.ds(base + nfull * CK, rem)],
                            part_hbm.at[cid, pl.ds(base + nfull * CK, rem)])

        @pl.when(sid == NS - 1)
        def _():
            pltpu.sync_copy(acc.at[pl.ds(tail_base, tail)],
                            part_hbm.at[cid, pl.ds(tail_base, tail)])

    return k(x, src4, dst4, w4)


def _sc_gather(light_tab, ego_tab, cat3):
    npc = cat3.shape[1]
    rows_pw = npc * CKE
    nrows = NW * rows_pw
    mesh = plsc.VectorSubcoreMesh(core_axis_name="c", subcore_axis_name="s")

    @functools.partial(
        pl.kernel,
        out_type=(
            jax.ShapeDtypeStruct((nrows, D), jnp.float32),
            jax.ShapeDtypeStruct((nrows, D), jnp.float32),
        ),
        mesh=mesh,
        scratch_types=[
            pltpu.VMEM((npc, CKE), jnp.int32),
            pltpu.VMEM((CKE, D), jnp.float32),
            pltpu.VMEM((CKE, D), jnp.float32),
            pltpu.SemaphoreType.DMA,
            pltpu.SemaphoreType.DMA,
        ],
    )
    def k(lt_hbm, et_hbm, idx_hbm, lo_hbm, eo_hbm, idx_v, bufa, bufb, sa, sb):
        cid = lax.axis_index("c")
        sid = lax.axis_index("s")
        wid = sid * NC + cid
        pltpu.sync_copy(idx_hbm.at[wid], idx_v)

        def chunk(j, _):
            ca = pltpu.async_copy(lt_hbm.at[idx_v.at[j]], bufa, sa)
            cb = pltpu.async_copy(et_hbm.at[idx_v.at[j]], bufb, sb)
            ca.wait()
            pltpu.sync_copy(bufa, lo_hbm.at[pl.ds(wid * rows_pw + j * CKE, CKE)])
            cb.wait()
            pltpu.sync_copy(bufb, eo_hbm.at[pl.ds(wid * rows_pw + j * CKE, CKE)])
            return 0
        lax.fori_loop(0, npc, chunk, 0)

    return k(light_tab, ego_tab, cat3)


def kernel(users, pos_items, neg_items, edge_index, edge_weight,
           user_sizes, item_sizes, user_table, item_table):
    ue = _masked(user_table, user_sizes)
    ie = _masked(item_table, item_sizes)
    x0 = jnp.concatenate([ue, ie], axis=0)

    src = edge_index[0].astype(jnp.int32)
    dst = edge_index[1].astype(jnp.int32)
    w = edge_weight.astype(jnp.float32)
    e = src.shape[0]
    unit = NW * SEC * CK * 2
    epad = ((e + unit - 1) // unit) * unit
    pad = epad - e
    nws = epad // (SEC * CK)
    src4 = jnp.pad(src, (0, pad)).reshape(nws, SEC, CK)
    dst4 = jnp.pad(dst, (0, pad)).reshape(nws, SEC, CK)
    w4 = jnp.pad(w, (0, pad)).reshape(nws, SEC, CK)

    x = x0
    s = x0
    for l in range(NLAYERS):
        part = _sc_layer(x, src4, dst4, w4)
        x, s = _combine(part, s, 0.25 if l == NLAYERS - 1 else 1.0)

    b = users.shape[0]
    cat = jnp.concatenate([
        users.astype(jnp.int32),
        pos_items.astype(jnp.int32) + NU,
        neg_items.astype(jnp.int32) + NU,
    ])
    cat3 = cat.reshape(NW, -1, CKE)
    light, ego = _sc_gather(s, x0, cat3)
    return (light[:b], light[b:2 * b], light[2 * b:],
            ego[:b], ego[b:2 * b], ego[2 * b:])


# speedup vs baseline: 2.0831x; 1.0475x over previous
import functools

import jax
import jax.numpy as jnp
from jax import lax
from jax.experimental import pallas as pl
from jax.experimental.pallas import tpu as pltpu
from jax.experimental.pallas import tpu_sc as plsc

NU = 5000
NI = 5000
NN = NU + NI
D = 128
NLAYERS = 3
NC, NS, LANES = 2, 16, 16
NW = NC * NS
CK = 112
CKE = 128
CPD = D // LANES


def _mask_body(tbl_ref, sz_ref, out_ref):
    it = lax.broadcasted_iota(jnp.int32, out_ref.shape, 1)
    out_ref[...] = tbl_ref[...] * (it < sz_ref[...]).astype(jnp.float32)


def _masked(tbl, sizes):
    return pl.pallas_call(
        _mask_body,
        out_shape=jax.ShapeDtypeStruct(tbl.shape, jnp.float32),
    )(tbl, sizes.astype(jnp.int32).reshape(-1, 1))


def _combine_body(scale):
    def body(p_ref, s_ref, y_ref, so_ref):
        y = p_ref[0] + p_ref[1]
        y_ref[...] = y
        so_ref[...] = (s_ref[...] + y) * scale
    return body


def _combine(part, s, scale):
    rb = NN // 10
    return pl.pallas_call(
        _combine_body(scale),
        grid=(10,),
        in_specs=[
            pl.BlockSpec((2, rb, D), lambda i: (0, i, 0)),
            pl.BlockSpec((rb, D), lambda i: (i, 0)),
        ],
        out_specs=[
            pl.BlockSpec((rb, D), lambda i: (i, 0)),
            pl.BlockSpec((rb, D), lambda i: (i, 0)),
        ],
        out_shape=[
            jax.ShapeDtypeStruct((NN, D), jnp.float32),
            jax.ShapeDtypeStruct((NN, D), jnp.float32),
        ],
    )(part, s)


SEC = 3
NBUF = 3


N0 = 32


def _sc_layer(x, src4, dst4, w4):
    nsec2 = src4.shape[0] // NS
    n1 = nsec2 - N0
    assert N0 % 2 == 0 and n1 % 2 == 0
    mesh = plsc.VectorSubcoreMesh(core_axis_name="c", subcore_axis_name="s")

    @functools.partial(
        pl.kernel,
        out_type=jax.ShapeDtypeStruct((NC, NN, D), jnp.float32),
        mesh=mesh,
        scratch_types=[
            [pltpu.VMEM((SEC, CK), jnp.int32) for _ in range(2)],
            [pltpu.VMEM((SEC, CK), jnp.int32) for _ in range(2)],
            [pltpu.VMEM((SEC, CK), jnp.float32) for _ in range(2)],
            [pltpu.VMEM((CK, D), jnp.float32) for _ in range(NBUF)],
            pltpu.VMEM_SHARED((NN, D), jnp.float32),
            [pltpu.SemaphoreType.DMA for _ in range(2)],
            [pltpu.SemaphoreType.DMA for _ in range(NBUF)],
            [pltpu.SemaphoreType.DMA for _ in range(NBUF)],
        ],
    )
    def k(x_hbm, src_hbm, dst_hbm, w_hbm, part_hbm,
          src_s, dst_s, w_s, bufs, acc, isem, gsem, ssem):
        cid = lax.axis_index("c")
        sid = lax.axis_index("s")
        sec_base = sid * nsec2 + cid * N0
        nsec = jnp.where(cid == 0, N0, n1)
        nct = nsec * SEC
        nround = nsec // 2

        zbuf = bufs[0]

        def zrow(r, _):
            for c in range(CPD):
                zbuf[r, pl.ds(c * LANES, LANES)] = jnp.zeros((LANES,), jnp.float32)
            return 0
        lax.fori_loop(0, CK, zrow, 0)

        rpt = 624
        nfull = rpt // CK
        rem = rpt - nfull * CK
        tail_base = NS * rpt
        tail = NN - tail_base
        base = sid * rpt
        for t in range(nfull):
            pltpu.sync_copy(zbuf, acc.at[pl.ds(base + t * CK, CK)])
        if rem:
            pltpu.sync_copy(zbuf.at[pl.ds(0, rem)],
                            acc.at[pl.ds(base + nfull * CK, rem)])

        @pl.when(sid == NS - 1)
        def _():
            pltpu.sync_copy(zbuf.at[pl.ds(0, tail)],
                            acc.at[pl.ds(tail_base, tail)])
        plsc.subcore_barrier()

        def load_sec(s, p):
            b = sec_base + s
            pltpu.async_copy(src_hbm.at[b], src_s[p], isem[p])
            pltpu.async_copy(dst_hbm.at[b], dst_s[p], isem[p])
            pltpu.async_copy(w_hbm.at[b], w_s[p], isem[p])

        def wait_sec(p):
            pltpu.make_async_copy(src_hbm.at[0], src_s[p], isem[p]).wait()
            pltpu.make_async_copy(dst_hbm.at[0], dst_s[p], isem[p]).wait()
            pltpu.make_async_copy(w_hbm.at[0], w_s[p], isem[p]).wait()

        def start_gather(p, jj, b):
            pltpu.async_copy(x_hbm.at[src_s[p].at[jj]], bufs[b], gsem[b])

        load_sec(0, 0)
        wait_sec(0)
        load_sec(1, 1)
        start_gather(0, 0, 0)
        start_gather(0, 1, 1)

        def rnd(t, _):
            for p in range(2):
                s = 2 * t + p

                @pl.when(s + 1 < nsec)
                def _():
                    wait_sec(1 - p)
                for jj in range(SEC):
                    c = 2 * SEC * t + SEC * p + jj
                    b = (SEC * p + jj) % NBUF
                    b2 = (b + 2) % NBUF
                    pltpu.make_async_copy(x_hbm.at[src_s[p].at[jj]],
                                          bufs[b], gsem[b]).wait()

                    @pl.when(c >= 1)
                    def _():
                        pltpu.make_async_copy(bufs[b2],
                                              acc.at[dst_s[p].at[jj]],
                                              ssem[b2]).wait()

                    @pl.when(c + 2 < nct)
                    def _():
                        if jj == 0:
                            start_gather(p, 2, b2)
                        else:
                            start_gather(1 - p, jj - 1, b2)

                    def grp_body(g, _, _b=b, _p=p, _jj=jj):
                        wgrp = w_s[_p][_jj, pl.ds(g * LANES, LANES)]
                        for lane in range(LANES):
                            wv = wgrp[lane]
                            r = g * LANES + lane
                            for cc in range(CPD):
                                bufs[_b][r, pl.ds(cc * LANES, LANES)] = (
                                    bufs[_b][r, pl.ds(cc * LANES, LANES)] * wv)
                        return 0
                    lax.fori_loop(0, CK // LANES, grp_body, 0)
                    pltpu.async_copy(bufs[b], acc.at[dst_s[p].at[jj]],
                                     ssem[b], add=True)

                @pl.when(s + 2 < nsec)
                def _():
                    load_sec(s + 2, p)
            return 0
        lax.fori_loop(0, nround, rnd, 0)

        bl = SEC - 1
        pl_last = 1
        pltpu.make_async_copy(bufs[bl], acc.at[dst_s[pl_last].at[SEC - 1]],
                              ssem[bl]).wait()
        plsc.subcore_barrier()

        for t in range(nfull):
            pltpu.sync_copy(acc.at[pl.ds(base + t * CK, CK)],
                            part_hbm.at[cid, pl.ds(base + t * CK, CK)])
        if rem:
            pltpu.sync_copy(acc.at[pl.ds(base + nfull * CK, rem)],
                            part_hbm.at[cid, pl.ds(base + nfull * CK, rem)])

        @pl.when(sid == NS - 1)
        def _():
            pltpu.sync_copy(acc.at[pl.ds(tail_base, tail)],
                            part_hbm.at[cid, pl.ds(tail_base, tail)])

    return k(x, src4, dst4, w4)


def _sc_gather(light_tab, ego_tab, cat3):
    npc = cat3.shape[1]
    rows_pw = npc * CKE
    nrows = NW * rows_pw
    mesh = plsc.VectorSubcoreMesh(core_axis_name="c", subcore_axis_name="s")

    @functools.partial(
        pl.kernel,
        out_type=(
            jax.ShapeDtypeStruct((nrows, D), jnp.float32),
            jax.ShapeDtypeStruct((nrows, D), jnp.float32),
        ),
        mesh=mesh,
        scratch_types=[
            pltpu.VMEM((npc, CKE), jnp.int32),
            pltpu.VMEM((CKE, D), jnp.float32),
            pltpu.VMEM((CKE, D), jnp.float32),
            pltpu.SemaphoreType.DMA,
            pltpu.SemaphoreType.DMA,
        ],
    )
    def k(lt_hbm, et_hbm, idx_hbm, lo_hbm, eo_hbm, idx_v, bufa, bufb, sa, sb):
        cid = lax.axis_index("c")
        sid = lax.axis_index("s")
        wid = sid * NC + cid
        pltpu.sync_copy(idx_hbm.at[wid], idx_v)

        def chunk(j, _):
            ca = pltpu.async_copy(lt_hbm.at[idx_v.at[j]], bufa, sa)
            cb = pltpu.async_copy(et_hbm.at[idx_v.at[j]], bufb, sb)
            ca.wait()
            pltpu.sync_copy(bufa, lo_hbm.at[pl.ds(wid * rows_pw + j * CKE, CKE)])
            cb.wait()
            pltpu.sync_copy(bufb, eo_hbm.at[pl.ds(wid * rows_pw + j * CKE, CKE)])
            return 0
        lax.fori_loop(0, npc, chunk, 0)

    return k(light_tab, ego_tab, cat3)


def kernel(users, pos_items, neg_items, edge_index, edge_weight,
           user_sizes, item_sizes, user_table, item_table):
    ue = _masked(user_table, user_sizes)
    ie = _masked(item_table, item_sizes)
    x0 = jnp.concatenate([ue, ie], axis=0)

    src = edge_index[0].astype(jnp.int32)
    dst = edge_index[1].astype(jnp.int32)
    w = edge_weight.astype(jnp.float32)
    e = src.shape[0]
    unit = NW * SEC * CK * 2
    epad = ((e + unit - 1) // unit) * unit
    pad = epad - e
    nws = epad // (SEC * CK)
    src4 = jnp.pad(src, (0, pad)).reshape(nws, SEC, CK)
    dst4 = jnp.pad(dst, (0, pad)).reshape(nws, SEC, CK)
    w4 = jnp.pad(w, (0, pad)).reshape(nws, SEC, CK)

    x = x0
    s = x0
    for l in range(NLAYERS):
        part = _sc_layer(x, src4, dst4, w4)
        x, s = _combine(part, s, 0.25 if l == NLAYERS - 1 else 1.0)

    b = users.shape[0]
    cat = jnp.concatenate([
        users.astype(jnp.int32),
        pos_items.astype(jnp.int32) + NU,
        neg_items.astype(jnp.int32) + NU,
    ])
    cat3 = cat.reshape(NW, -1, CKE)
    light, ego = _sc_gather(s, x0, cat3)
    return (light[:b], light[b:2 * b], light[2 * b:],
            ego[:b], ego[b:2 * b], ego[2 * b:])


# speedup vs baseline: 2.1372x; 1.0260x over previous
import functools

import jax
import jax.numpy as jnp
from jax import lax
from jax.experimental import pallas as pl
from jax.experimental.pallas import tpu as pltpu
from jax.experimental.pallas import tpu_sc as plsc

NU = 5000
NI = 5000
NN = NU + NI
D = 128
NLAYERS = 3
NC, NS, LANES = 2, 16, 16
NW = NC * NS
CK = 112
CKE = 128
CPD = D // LANES


def _mask_body(tbl_ref, sz_ref, out_ref):
    it = lax.broadcasted_iota(jnp.int32, out_ref.shape, 1)
    out_ref[...] = tbl_ref[...] * (it < sz_ref[...]).astype(jnp.float32)


def _masked(tbl, sizes):
    return pl.pallas_call(
        _mask_body,
        out_shape=jax.ShapeDtypeStruct(tbl.shape, jnp.float32),
    )(tbl, sizes.astype(jnp.int32).reshape(-1, 1))


def _combine_body(scale):
    def body(p_ref, s_ref, y_ref, so_ref):
        y = p_ref[0] + p_ref[1]
        y_ref[...] = y
        so_ref[...] = (s_ref[...] + y) * scale
    return body


def _combine(part, s, scale):
    rb = NN // 10
    return pl.pallas_call(
        _combine_body(scale),
        grid=(10,),
        in_specs=[
            pl.BlockSpec((2, rb, D), lambda i: (0, i, 0)),
            pl.BlockSpec((rb, D), lambda i: (i, 0)),
        ],
        out_specs=[
            pl.BlockSpec((rb, D), lambda i: (i, 0)),
            pl.BlockSpec((rb, D), lambda i: (i, 0)),
        ],
        out_shape=[
            jax.ShapeDtypeStruct((NN, D), jnp.float32),
            jax.ShapeDtypeStruct((NN, D), jnp.float32),
        ],
    )(part, s)


SEC = 3
NBUF = 3


N0 = 34


def _sc_layer(x, src4, dst4, w4):
    nsec2 = src4.shape[0] // NS
    n1 = nsec2 - N0
    assert N0 % 2 == 0 and n1 % 2 == 0
    mesh = plsc.VectorSubcoreMesh(core_axis_name="c", subcore_axis_name="s")

    @functools.partial(
        pl.kernel,
        out_type=jax.ShapeDtypeStruct((NC, NN, D), jnp.float32),
        mesh=mesh,
        scratch_types=[
            [pltpu.VMEM((SEC, CK), jnp.int32) for _ in range(2)],
            [pltpu.VMEM((SEC, CK), jnp.int32) for _ in range(2)],
            [pltpu.VMEM((SEC, CK), jnp.float32) for _ in range(2)],
            [pltpu.VMEM((CK, D), jnp.float32) for _ in range(NBUF)],
            pltpu.VMEM_SHARED((NN, D), jnp.float32),
            [pltpu.SemaphoreType.DMA for _ in range(2)],
            [pltpu.SemaphoreType.DMA for _ in range(NBUF)],
            [pltpu.SemaphoreType.DMA for _ in range(NBUF)],
        ],
    )
    def k(x_hbm, src_hbm, dst_hbm, w_hbm, part_hbm,
          src_s, dst_s, w_s, bufs, acc, isem, gsem, ssem):
        cid = lax.axis_index("c")
        sid = lax.axis_index("s")
        sec_base = sid * nsec2 + cid * N0
        nsec = jnp.where(cid == 0, N0, n1)
        nct = nsec * SEC
        nround = nsec // 2

        zbuf = bufs[0]

        def zrow(r, _):
            for c in range(CPD):
                zbuf[r, pl.ds(c * LANES, LANES)] = jnp.zeros((LANES,), jnp.float32)
            return 0
        lax.fori_loop(0, CK, zrow, 0)

        rpt = 624
        nfull = rpt // CK
        rem = rpt - nfull * CK
        tail_base = NS * rpt
        tail = NN - tail_base
        base = sid * rpt
        for t in range(nfull):
            pltpu.sync_copy(zbuf, acc.at[pl.ds(base + t * CK, CK)])
        if rem:
            pltpu.sync_copy(zbuf.at[pl.ds(0, rem)],
                            acc.at[pl.ds(base + nfull * CK, rem)])

        @pl.when(sid == NS - 1)
        def _():
            pltpu.sync_copy(zbuf.at[pl.ds(0, tail)],
                            acc.at[pl.ds(tail_base, tail)])
        plsc.subcore_barrier()

        def load_sec(s, p):
            b = sec_base + s
            pltpu.async_copy(src_hbm.at[b], src_s[p], isem[p])
            pltpu.async_copy(dst_hbm.at[b], dst_s[p], isem[p])
            pltpu.async_copy(w_hbm.at[b], w_s[p], isem[p])

        def wait_sec(p):
            pltpu.make_async_copy(src_hbm.at[0], src_s[p], isem[p]).wait()
            pltpu.make_async_copy(dst_hbm.at[0], dst_s[p], isem[p]).wait()
            pltpu.make_async_copy(w_hbm.at[0], w_s[p], isem[p]).wait()

        def start_gather(p, jj, b):
            pltpu.async_copy(x_hbm.at[src_s[p].at[jj]], bufs[b], gsem[b])

        load_sec(0, 0)
        wait_sec(0)
        load_sec(1, 1)
        start_gather(0, 0, 0)
        start_gather(0, 1, 1)

        def rnd(t, _):
            for p in range(2):
                s = 2 * t + p

                @pl.when(s + 1 < nsec)
                def _():
                    wait_sec(1 - p)
                for jj in range(SEC):
                    c = 2 * SEC * t + SEC * p + jj
                    b = (SEC * p + jj) % NBUF
                    b2 = (b + 2) % NBUF
                    pltpu.make_async_copy(x_hbm.at[src_s[p].at[jj]],
                                          bufs[b], gsem[b]).wait()

                    @pl.when(c >= 1)
                    def _():
                        pltpu.make_async_copy(bufs[b2],
                                              acc.at[dst_s[p].at[jj]],
                                              ssem[b2]).wait()

                    @pl.when(c + 2 < nct)
                    def _():
                        if jj == 0:
                            start_gather(p, 2, b2)
                        else:
                            start_gather(1 - p, jj - 1, b2)

                    def grp_body(g, _, _b=b, _p=p, _jj=jj):
                        wgrp = w_s[_p][_jj, pl.ds(g * LANES, LANES)]
                        for lane in range(LANES):
                            wv = wgrp[lane]
                            r = g * LANES + lane
                            for cc in range(CPD):
                                bufs[_b][r, pl.ds(cc * LANES, LANES)] = (
                                    bufs[_b][r, pl.ds(cc * LANES, LANES)] * wv)
                        return 0
                    lax.fori_loop(0, CK // LANES, grp_body, 0)
                    pltpu.async_copy(bufs[b], acc.at[dst_s[p].at[jj]],
                                     ssem[b], add=True)

                @pl.when(s + 2 < nsec)
                def _():
                    load_sec(s + 2, p)
            return 0
        lax.fori_loop(0, nround, rnd, 0)

        bl = SEC - 1
        pl_last = 1
        pltpu.make_async_copy(bufs[bl], acc.at[dst_s[pl_last].at[SEC - 1]],
                              ssem[bl]).wait()
        plsc.subcore_barrier()

        for t in range(nfull):
            pltpu.sync_copy(acc.at[pl.ds(base + t * CK, CK)],
                            part_hbm.at[cid, pl.ds(base + t * CK, CK)])
        if rem:
            pltpu.sync_copy(acc.at[pl.ds(base + nfull * CK, rem)],
                            part_hbm.at[cid, pl.ds(base + nfull * CK, rem)])

        @pl.when(sid == NS - 1)
        def _():
            pltpu.sync_copy(acc.at[pl.ds(tail_base, tail)],
                            part_hbm.at[cid, pl.ds(tail_base, tail)])

    return k(x, src4, dst4, w4)


def _sc_gather(light_tab, ego_tab, cat3):
    npc = cat3.shape[1]
    rows_pw = npc * CKE
    nrows = NW * rows_pw
    mesh = plsc.VectorSubcoreMesh(core_axis_name="c", subcore_axis_name="s")

    @functools.partial(
        pl.kernel,
        out_type=(
            jax.ShapeDtypeStruct((nrows, D), jnp.float32),
            jax.ShapeDtypeStruct((nrows, D), jnp.float32),
        ),
        mesh=mesh,
        scratch_types=[
            pltpu.VMEM((npc, CKE), jnp.int32),
            pltpu.VMEM((CKE, D), jnp.float32),
            pltpu.VMEM((CKE, D), jnp.float32),
            pltpu.SemaphoreType.DMA,
            pltpu.SemaphoreType.DMA,
        ],
    )
    def k(lt_hbm, et_hbm, idx_hbm, lo_hbm, eo_hbm, idx_v, bufa, bufb, sa, sb):
        cid = lax.axis_index("c")
        sid = lax.axis_index("s")
        wid = sid * NC + cid
        pltpu.sync_copy(idx_hbm.at[wid], idx_v)

        def chunk(j, _):
            ca = pltpu.async_copy(lt_hbm.at[idx_v.at[j]], bufa, sa)
            cb = pltpu.async_copy(et_hbm.at[idx_v.at[j]], bufb, sb)
            ca.wait()
            pltpu.sync_copy(bufa, lo_hbm.at[pl.ds(wid * rows_pw + j * CKE, CKE)])
            cb.wait()
            pltpu.sync_copy(bufb, eo_hbm.at[pl.ds(wid * rows_pw + j * CKE, CKE)])
            return 0
        lax.fori_loop(0, npc, chunk, 0)

    return k(light_tab, ego_tab, cat3)


def kernel(users, pos_items, neg_items, edge_index, edge_weight,
           user_sizes, item_sizes, user_table, item_table):
    ue = _masked(user_table, user_sizes)
    ie = _masked(item_table, item_sizes)
    x0 = jnp.concatenate([ue, ie], axis=0)

    src = edge_index[0].astype(jnp.int32)
    dst = edge_index[1].astype(jnp.int32)
    w = edge_weight.astype(jnp.float32)
    e = src.shape[0]
    unit = NW * SEC * CK * 2
    epad = ((e + unit - 1) // unit) * unit
    pad = epad - e
    nws = epad // (SEC * CK)
    src4 = jnp.pad(src, (0, pad)).reshape(nws, SEC, CK)
    dst4 = jnp.pad(dst, (0, pad)).reshape(nws, SEC, CK)
    w4 = jnp.pad(w, (0, pad)).reshape(nws, SEC, CK)

    x = x0
    s = x0
    for l in range(NLAYERS):
        part = _sc_layer(x, src4, dst4, w4)
        x, s = _combine(part, s, 0.25 if l == NLAYERS - 1 else 1.0)

    b = users.shape[0]
    cat = jnp.concatenate([
        users.astype(jnp.int32),
        pos_items.astype(jnp.int32) + NU,
        neg_items.astype(jnp.int32) + NU,
    ])
    cat3 = cat.reshape(NW, -1, CKE)
    light, ego = _sc_gather(s, x0, cat3)
    return (light[:b], light[b:2 * b], light[2 * b:],
            ego[:b], ego[b:2 * b], ego[2 * b:])


# speedup vs baseline: 2.1989x; 1.0288x over previous
import functools

import jax
import jax.numpy as jnp
from jax import lax
from jax.experimental import pallas as pl
from jax.experimental.pallas import tpu as pltpu
from jax.experimental.pallas import tpu_sc as plsc

NU = 5000
NI = 5000
NN = NU + NI
D = 128
NLAYERS = 3
NC, NS, LANES = 2, 16, 16
NW = NC * NS
CK = 112
CKE = 128
CPD = D // LANES


def _mask_body(tbl_ref, sz_ref, out_ref):
    it = lax.broadcasted_iota(jnp.int32, out_ref.shape, 1)
    out_ref[...] = tbl_ref[...] * (it < sz_ref[...]).astype(jnp.float32)


def _masked(tbl, sizes):
    return pl.pallas_call(
        _mask_body,
        out_shape=jax.ShapeDtypeStruct(tbl.shape, jnp.float32),
    )(tbl, sizes.astype(jnp.int32).reshape(-1, 1))


def _combine_body(scale):
    def body(p_ref, s_ref, y_ref, so_ref):
        y = p_ref[0] + p_ref[1]
        y_ref[...] = y
        so_ref[...] = (s_ref[...] + y) * scale
    return body


def _combine(part, s, scale):
    rb = NN // 10
    return pl.pallas_call(
        _combine_body(scale),
        grid=(10,),
        in_specs=[
            pl.BlockSpec((2, rb, D), lambda i: (0, i, 0)),
            pl.BlockSpec((rb, D), lambda i: (i, 0)),
        ],
        out_specs=[
            pl.BlockSpec((rb, D), lambda i: (i, 0)),
            pl.BlockSpec((rb, D), lambda i: (i, 0)),
        ],
        out_shape=[
            jax.ShapeDtypeStruct((NN, D), jnp.float32),
            jax.ShapeDtypeStruct((NN, D), jnp.float32),
        ],
    )(part, s)


SEC = 3
NBUF = 3


N0 = 36


def _sc_layer(x, src4, dst4, w4):
    nsec2 = src4.shape[0] // NS
    n1 = nsec2 - N0
    assert N0 % 2 == 0 and n1 % 2 == 0
    mesh = plsc.VectorSubcoreMesh(core_axis_name="c", subcore_axis_name="s")

    @functools.partial(
        pl.kernel,
        out_type=jax.ShapeDtypeStruct((NC, NN, D), jnp.float32),
        mesh=mesh,
        scratch_types=[
            [pltpu.VMEM((SEC, CK), jnp.int32) for _ in range(2)],
            [pltpu.VMEM((SEC, CK), jnp.int32) for _ in range(2)],
            [pltpu.VMEM((SEC, CK), jnp.float32) for _ in range(2)],
            [pltpu.VMEM((CK, D), jnp.float32) for _ in range(NBUF)],
            pltpu.VMEM_SHARED((NN, D), jnp.float32),
            [pltpu.SemaphoreType.DMA for _ in range(2)],
            [pltpu.SemaphoreType.DMA for _ in range(NBUF)],
            [pltpu.SemaphoreType.DMA for _ in range(NBUF)],
        ],
    )
    def k(x_hbm, src_hbm, dst_hbm, w_hbm, part_hbm,
          src_s, dst_s, w_s, bufs, acc, isem, gsem, ssem):
        cid = lax.axis_index("c")
        sid = lax.axis_index("s")
        sec_base = sid * nsec2 + cid * N0
        nsec = jnp.where(cid == 0, N0, n1)
        nct = nsec * SEC
        nround = nsec // 2

        zbuf = bufs[0]

        def zrow(r, _):
            for c in range(CPD):
                zbuf[r, pl.ds(c * LANES, LANES)] = jnp.zeros((LANES,), jnp.float32)
            return 0
        lax.fori_loop(0, CK, zrow, 0)

        rpt = 624
        nfull = rpt // CK
        rem = rpt - nfull * CK
        tail_base = NS * rpt
        tail = NN - tail_base
        base = sid * rpt
        for t in range(nfull):
            pltpu.sync_copy(zbuf, acc.at[pl.ds(base + t * CK, CK)])
        if rem:
            pltpu.sync_copy(zbuf.at[pl.ds(0, rem)],
                            acc.at[pl.ds(base + nfull * CK, rem)])

        @pl.when(sid == NS - 1)
        def _():
            pltpu.sync_copy(zbuf.at[pl.ds(0, tail)],
                            acc.at[pl.ds(tail_base, tail)])
        plsc.subcore_barrier()

        def load_sec(s, p):
            b = sec_base + s
            pltpu.async_copy(src_hbm.at[b], src_s[p], isem[p])
            pltpu.async_copy(dst_hbm.at[b], dst_s[p], isem[p])
            pltpu.async_copy(w_hbm.at[b], w_s[p], isem[p])

        def wait_sec(p):
            pltpu.make_async_copy(src_hbm.at[0], src_s[p], isem[p]).wait()
            pltpu.make_async_copy(dst_hbm.at[0], dst_s[p], isem[p]).wait()
            pltpu.make_async_copy(w_hbm.at[0], w_s[p], isem[p]).wait()

        def start_gather(p, jj, b):
            pltpu.async_copy(x_hbm.at[src_s[p].at[jj]], bufs[b], gsem[b])

        load_sec(0, 0)
        wait_sec(0)
        load_sec(1, 1)
        start_gather(0, 0, 0)
        start_gather(0, 1, 1)

        def rnd(t, _):
            for p in range(2):
                s = 2 * t + p

                @pl.when(s + 1 < nsec)
                def _():
                    wait_sec(1 - p)
                for jj in range(SEC):
                    c = 2 * SEC * t + SEC * p + jj
                    b = (SEC * p + jj) % NBUF
                    b2 = (b + 2) % NBUF
                    pltpu.make_async_copy(x_hbm.at[src_s[p].at[jj]],
                                          bufs[b], gsem[b]).wait()

                    @pl.when(c >= 1)
                    def _():
                        pltpu.make_async_copy(bufs[b2],
                                              acc.at[dst_s[p].at[jj]],
                                              ssem[b2]).wait()

                    @pl.when(c + 2 < nct)
                    def _():
                        if jj == 0:
                            start_gather(p, 2, b2)
                        else:
                            start_gather(1 - p, jj - 1, b2)

                    def grp_body(g, _, _b=b, _p=p, _jj=jj):
                        wgrp = w_s[_p][_jj, pl.ds(g * LANES, LANES)]
                        for lane in range(LANES):
                            wv = wgrp[lane]
                            r = g * LANES + lane
                            for cc in range(CPD):
                                bufs[_b][r, pl.ds(cc * LANES, LANES)] = (
                                    bufs[_b][r, pl.ds(cc * LANES, LANES)] * wv)
                        return 0
                    lax.fori_loop(0, CK // LANES, grp_body, 0)
                    pltpu.async_copy(bufs[b], acc.at[dst_s[p].at[jj]],
                                     ssem[b], add=True)

                @pl.when(s + 2 < nsec)
                def _():
                    load_sec(s + 2, p)
            return 0
        lax.fori_loop(0, nround, rnd, 0)

        bl = SEC - 1
        pl_last = 1
        pltpu.make_async_copy(bufs[bl], acc.at[dst_s[pl_last].at[SEC - 1]],
                              ssem[bl]).wait()
        plsc.subcore_barrier()

        for t in range(nfull):
            pltpu.sync_copy(acc.at[pl.ds(base + t * CK, CK)],
                            part_hbm.at[cid, pl.ds(base + t * CK, CK)])
        if rem:
            pltpu.sync_copy(acc.at[pl.ds(base + nfull * CK, rem)],
                            part_hbm.at[cid, pl.ds(base + nfull * CK, rem)])

        @pl.when(sid == NS - 1)
        def _():
            pltpu.sync_copy(acc.at[pl.ds(tail_base, tail)],
                            part_hbm.at[cid, pl.ds(tail_base, tail)])

    return k(x, src4, dst4, w4)


def _sc_gather(light_tab, ego_tab, cat3):
    npc = cat3.shape[1]
    rows_pw = npc * CKE
    nrows = NW * rows_pw
    mesh = plsc.VectorSubcoreMesh(core_axis_name="c", subcore_axis_name="s")

    @functools.partial(
        pl.kernel,
        out_type=(
            jax.ShapeDtypeStruct((nrows, D), jnp.float32),
            jax.ShapeDtypeStruct((nrows, D), jnp.float32),
        ),
        mesh=mesh,
        scratch_types=[
            pltpu.VMEM((npc, CKE), jnp.int32),
            pltpu.VMEM((CKE, D), jnp.float32),
            pltpu.VMEM((CKE, D), jnp.float32),
            pltpu.SemaphoreType.DMA,
            pltpu.SemaphoreType.DMA,
        ],
    )
    def k(lt_hbm, et_hbm, idx_hbm, lo_hbm, eo_hbm, idx_v, bufa, bufb, sa, sb):
        cid = lax.axis_index("c")
        sid = lax.axis_index("s")
        wid = sid * NC + cid
        pltpu.sync_copy(idx_hbm.at[wid], idx_v)

        def chunk(j, _):
            ca = pltpu.async_copy(lt_hbm.at[idx_v.at[j]], bufa, sa)
            cb = pltpu.async_copy(et_hbm.at[idx_v.at[j]], bufb, sb)
            ca.wait()
            pltpu.sync_copy(bufa, lo_hbm.at[pl.ds(wid * rows_pw + j * CKE, CKE)])
            cb.wait()
            pltpu.sync_copy(bufb, eo_hbm.at[pl.ds(wid * rows_pw + j * CKE, CKE)])
            return 0
        lax.fori_loop(0, npc, chunk, 0)

    return k(light_tab, ego_tab, cat3)


def kernel(users, pos_items, neg_items, edge_index, edge_weight,
           user_sizes, item_sizes, user_table, item_table):
    ue = _masked(user_table, user_sizes)
    ie = _masked(item_table, item_sizes)
    x0 = jnp.concatenate([ue, ie], axis=0)

    src = edge_index[0].astype(jnp.int32)
    dst = edge_index[1].astype(jnp.int32)
    w = edge_weight.astype(jnp.float32)
    e = src.shape[0]
    unit = NW * SEC * CK * 2
    epad = ((e + unit - 1) // unit) * unit
    pad = epad - e
    nws = epad // (SEC * CK)
    src4 = jnp.pad(src, (0, pad)).reshape(nws, SEC, CK)
    dst4 = jnp.pad(dst, (0, pad)).reshape(nws, SEC, CK)
    w4 = jnp.pad(w, (0, pad)).reshape(nws, SEC, CK)

    x = x0
    s = x0
    for l in range(NLAYERS):
        part = _sc_layer(x, src4, dst4, w4)
        x, s = _combine(part, s, 0.25 if l == NLAYERS - 1 else 1.0)

    b = users.shape[0]
    cat = jnp.concatenate([
        users.astype(jnp.int32),
        pos_items.astype(jnp.int32) + NU,
        neg_items.astype(jnp.int32) + NU,
    ])
    cat3 = cat.reshape(NW, -1, CKE)
    light, ego = _sc_gather(s, x0, cat3)
    return (light[:b], light[b:2 * b], light[2 * b:],
            ego[:b], ego[b:2 * b], ego[2 * b:])


# speedup vs baseline: 2.3481x; 1.0679x over previous
import functools

import jax
import jax.numpy as jnp
from jax import lax
from jax.experimental import pallas as pl
from jax.experimental.pallas import tpu as pltpu
from jax.experimental.pallas import tpu_sc as plsc

NU = 5000
NI = 5000
NN = NU + NI
D = 128
NLAYERS = 3
NC, NS, LANES = 2, 16, 16
NW = NC * NS
CK = 112
CKE = 128
CPD = D // LANES


def _mask_body(tbl_ref, sz_ref, out_ref):
    it = lax.broadcasted_iota(jnp.int32, out_ref.shape, 1)
    out_ref[...] = tbl_ref[...] * (it < sz_ref[...]).astype(jnp.float32)


def _masked(tbl, sizes):
    return pl.pallas_call(
        _mask_body,
        out_shape=jax.ShapeDtypeStruct(tbl.shape, jnp.float32),
    )(tbl, sizes.astype(jnp.int32).reshape(-1, 1))


def _combine_body(scale):
    def body(p_ref, s_ref, y_ref, so_ref):
        y = p_ref[0] + p_ref[1]
        y_ref[...] = y
        so_ref[...] = (s_ref[...] + y) * scale
    return body


def _combine(part, s, scale):
    rb = NN // 10
    return pl.pallas_call(
        _combine_body(scale),
        grid=(10,),
        in_specs=[
            pl.BlockSpec((2, rb, D), lambda i: (0, i, 0)),
            pl.BlockSpec((rb, D), lambda i: (i, 0)),
        ],
        out_specs=[
            pl.BlockSpec((rb, D), lambda i: (i, 0)),
            pl.BlockSpec((rb, D), lambda i: (i, 0)),
        ],
        out_shape=[
            jax.ShapeDtypeStruct((NN, D), jnp.float32),
            jax.ShapeDtypeStruct((NN, D), jnp.float32),
        ],
    )(part, s)


SEC = 3
NBUF = 3


N0 = 40


def _sc_layer(x, src4, dst4, w4):
    nsec2 = src4.shape[0] // NS
    n1 = nsec2 - N0
    assert N0 % 2 == 0 and n1 % 2 == 0
    mesh = plsc.VectorSubcoreMesh(core_axis_name="c", subcore_axis_name="s")

    @functools.partial(
        pl.kernel,
        out_type=jax.ShapeDtypeStruct((NC, NN, D), jnp.float32),
        mesh=mesh,
        scratch_types=[
            [pltpu.VMEM((SEC, CK), jnp.int32) for _ in range(2)],
            [pltpu.VMEM((SEC, CK), jnp.int32) for _ in range(2)],
            [pltpu.VMEM((SEC, CK), jnp.float32) for _ in range(2)],
            [pltpu.VMEM((CK, D), jnp.float32) for _ in range(NBUF)],
            pltpu.VMEM_SHARED((NN, D), jnp.float32),
            [pltpu.SemaphoreType.DMA for _ in range(2)],
            [pltpu.SemaphoreType.DMA for _ in range(NBUF)],
            [pltpu.SemaphoreType.DMA for _ in range(NBUF)],
        ],
    )
    def k(x_hbm, src_hbm, dst_hbm, w_hbm, part_hbm,
          src_s, dst_s, w_s, bufs, acc, isem, gsem, ssem):
        cid = lax.axis_index("c")
        sid = lax.axis_index("s")
        sec_base = sid * nsec2 + cid * N0
        nsec = jnp.where(cid == 0, N0, n1)
        nct = nsec * SEC
        nround = nsec // 2

        zbuf = bufs[0]

        def zrow(r, _):
            for c in range(CPD):
                zbuf[r, pl.ds(c * LANES, LANES)] = jnp.zeros((LANES,), jnp.float32)
            return 0
        lax.fori_loop(0, CK, zrow, 0)

        rpt = 624
        nfull = rpt // CK
        rem = rpt - nfull * CK
        tail_base = NS * rpt
        tail = NN - tail_base
        base = sid * rpt
        for t in range(nfull):
            pltpu.sync_copy(zbuf, acc.at[pl.ds(base + t * CK, CK)])
        if rem:
            pltpu.sync_copy(zbuf.at[pl.ds(0, rem)],
                            acc.at[pl.ds(base + nfull * CK, rem)])

        @pl.when(sid == NS - 1)
        def _():
            pltpu.sync_copy(zbuf.at[pl.ds(0, tail)],
                            acc.at[pl.ds(tail_base, tail)])
        plsc.subcore_barrier()

        def load_sec(s, p):
            b = sec_base + s
            pltpu.async_copy(src_hbm.at[b], src_s[p], isem[p])
            pltpu.async_copy(dst_hbm.at[b], dst_s[p], isem[p])
            pltpu.async_copy(w_hbm.at[b], w_s[p], isem[p])

        def wait_sec(p):
            pltpu.make_async_copy(src_hbm.at[0], src_s[p], isem[p]).wait()
            pltpu.make_async_copy(dst_hbm.at[0], dst_s[p], isem[p]).wait()
            pltpu.make_async_copy(w_hbm.at[0], w_s[p], isem[p]).wait()

        def start_gather(p, jj, b):
            pltpu.async_copy(x_hbm.at[src_s[p].at[jj]], bufs[b], gsem[b])

        load_sec(0, 0)
        wait_sec(0)
        load_sec(1, 1)
        start_gather(0, 0, 0)
        start_gather(0, 1, 1)

        def rnd(t, _):
            for p in range(2):
                s = 2 * t + p

                @pl.when(s + 1 < nsec)
                def _():
                    wait_sec(1 - p)
                for jj in range(SEC):
                    c = 2 * SEC * t + SEC * p + jj
                    b = (SEC * p + jj) % NBUF
                    b2 = (b + 2) % NBUF
                    pltpu.make_async_copy(x_hbm.at[src_s[p].at[jj]],
                                          bufs[b], gsem[b]).wait()

                    @pl.when(c >= 1)
                    def _():
                        pltpu.make_async_copy(bufs[b2],
                                              acc.at[dst_s[p].at[jj]],
                                              ssem[b2]).wait()

                    @pl.when(c + 2 < nct)
                    def _():
                        if jj == 0:
                            start_gather(p, 2, b2)
                        else:
                            start_gather(1 - p, jj - 1, b2)

                    def grp_body(g, _, _b=b, _p=p, _jj=jj):
                        wgrp = w_s[_p][_jj, pl.ds(g * LANES, LANES)]
                        for lane in range(LANES):
                            wv = wgrp[lane]
                            r = g * LANES + lane
                            for cc in range(CPD):
                                bufs[_b][r, pl.ds(cc * LANES, LANES)] = (
                                    bufs[_b][r, pl.ds(cc * LANES, LANES)] * wv)
                        return 0
                    lax.fori_loop(0, CK // LANES, grp_body, 0)
                    pltpu.async_copy(bufs[b], acc.at[dst_s[p].at[jj]],
                                     ssem[b], add=True)

                @pl.when(s + 2 < nsec)
                def _():
                    load_sec(s + 2, p)
            return 0
        lax.fori_loop(0, nround, rnd, 0)

        bl = SEC - 1
        pl_last = 1
        pltpu.make_async_copy(bufs[bl], acc.at[dst_s[pl_last].at[SEC - 1]],
                              ssem[bl]).wait()
        plsc.subcore_barrier()

        for t in range(nfull):
            pltpu.sync_copy(acc.at[pl.ds(base + t * CK, CK)],
                            part_hbm.at[cid, pl.ds(base + t * CK, CK)])
        if rem:
            pltpu.sync_copy(acc.at[pl.ds(base + nfull * CK, rem)],
                            part_hbm.at[cid, pl.ds(base + nfull * CK, rem)])

        @pl.when(sid == NS - 1)
        def _():
            pltpu.sync_copy(acc.at[pl.ds(tail_base, tail)],
                            part_hbm.at[cid, pl.ds(tail_base, tail)])

    return k(x, src4, dst4, w4)


def _sc_gather(light_tab, ego_tab, cat3):
    npc = cat3.shape[1]
    rows_pw = npc * CKE
    nrows = NW * rows_pw
    mesh = plsc.VectorSubcoreMesh(core_axis_name="c", subcore_axis_name="s")

    @functools.partial(
        pl.kernel,
        out_type=(
            jax.ShapeDtypeStruct((nrows, D), jnp.float32),
            jax.ShapeDtypeStruct((nrows, D), jnp.float32),
        ),
        mesh=mesh,
        scratch_types=[
            pltpu.VMEM((npc, CKE), jnp.int32),
            pltpu.VMEM((CKE, D), jnp.float32),
            pltpu.VMEM((CKE, D), jnp.float32),
            pltpu.SemaphoreType.DMA,
            pltpu.SemaphoreType.DMA,
        ],
    )
    def k(lt_hbm, et_hbm, idx_hbm, lo_hbm, eo_hbm, idx_v, bufa, bufb, sa, sb):
        cid = lax.axis_index("c")
        sid = lax.axis_index("s")
        wid = sid * NC + cid
        pltpu.sync_copy(idx_hbm.at[wid], idx_v)

        def chunk(j, _):
            ca = pltpu.async_copy(lt_hbm.at[idx_v.at[j]], bufa, sa)
            cb = pltpu.async_copy(et_hbm.at[idx_v.at[j]], bufb, sb)
            ca.wait()
            pltpu.sync_copy(bufa, lo_hbm.at[pl.ds(wid * rows_pw + j * CKE, CKE)])
            cb.wait()
            pltpu.sync_copy(bufb, eo_hbm.at[pl.ds(wid * rows_pw + j * CKE, CKE)])
            return 0
        lax.fori_loop(0, npc, chunk, 0)

    return k(light_tab, ego_tab, cat3)


def kernel(users, pos_items, neg_items, edge_index, edge_weight,
           user_sizes, item_sizes, user_table, item_table):
    ue = _masked(user_table, user_sizes)
    ie = _masked(item_table, item_sizes)
    x0 = jnp.concatenate([ue, ie], axis=0)

    src = edge_index[0].astype(jnp.int32)
    dst = edge_index[1].astype(jnp.int32)
    w = edge_weight.astype(jnp.float32)
    e = src.shape[0]
    unit = NW * SEC * CK * 2
    epad = ((e + unit - 1) // unit) * unit
    pad = epad - e
    nws = epad // (SEC * CK)
    src4 = jnp.pad(src, (0, pad)).reshape(nws, SEC, CK)
    dst4 = jnp.pad(dst, (0, pad)).reshape(nws, SEC, CK)
    w4 = jnp.pad(w, (0, pad)).reshape(nws, SEC, CK)

    x = x0
    s = x0
    for l in range(NLAYERS):
        part = _sc_layer(x, src4, dst4, w4)
        x, s = _combine(part, s, 0.25 if l == NLAYERS - 1 else 1.0)

    b = users.shape[0]
    cat = jnp.concatenate([
        users.astype(jnp.int32),
        pos_items.astype(jnp.int32) + NU,
        neg_items.astype(jnp.int32) + NU,
    ])
    cat3 = cat.reshape(NW, -1, CKE)
    light, ego = _sc_gather(s, x0, cat3)
    return (light[:b], light[b:2 * b], light[2 * b:],
            ego[:b], ego[b:2 * b], ego[2 * b:])


# speedup vs baseline: 2.4805x; 1.0564x over previous
import functools

import jax
import jax.numpy as jnp
from jax import lax
from jax.experimental import pallas as pl
from jax.experimental.pallas import tpu as pltpu
from jax.experimental.pallas import tpu_sc as plsc

NU = 5000
NI = 5000
NN = NU + NI
D = 128
NLAYERS = 3
NC, NS, LANES = 2, 16, 16
NW = NC * NS
CK = 112
CKE = 128
CPD = D // LANES


def _mask_body(tbl_ref, sz_ref, out_ref):
    it = lax.broadcasted_iota(jnp.int32, out_ref.shape, 1)
    out_ref[...] = tbl_ref[...] * (it < sz_ref[...]).astype(jnp.float32)


def _masked(tbl, sizes):
    return pl.pallas_call(
        _mask_body,
        out_shape=jax.ShapeDtypeStruct(tbl.shape, jnp.float32),
    )(tbl, sizes.astype(jnp.int32).reshape(-1, 1))


def _combine_body(scale):
    def body(p_ref, s_ref, y_ref, so_ref):
        y = p_ref[0] + p_ref[1]
        y_ref[...] = y
        so_ref[...] = (s_ref[...] + y) * scale
    return body


def _combine(part, s, scale):
    rb = NN // 10
    return pl.pallas_call(
        _combine_body(scale),
        grid=(10,),
        in_specs=[
            pl.BlockSpec((2, rb, D), lambda i: (0, i, 0)),
            pl.BlockSpec((rb, D), lambda i: (i, 0)),
        ],
        out_specs=[
            pl.BlockSpec((rb, D), lambda i: (i, 0)),
            pl.BlockSpec((rb, D), lambda i: (i, 0)),
        ],
        out_shape=[
            jax.ShapeDtypeStruct((NN, D), jnp.float32),
            jax.ShapeDtypeStruct((NN, D), jnp.float32),
        ],
    )(part, s)


SEC = 3
NBUF = 3


N0 = 44


def _sc_layer(x, src4, dst4, w4):
    nsec2 = src4.shape[0] // NS
    n1 = nsec2 - N0
    assert N0 % 2 == 0 and n1 % 2 == 0
    mesh = plsc.VectorSubcoreMesh(core_axis_name="c", subcore_axis_name="s")

    @functools.partial(
        pl.kernel,
        out_type=jax.ShapeDtypeStruct((NC, NN, D), jnp.float32),
        mesh=mesh,
        scratch_types=[
            [pltpu.VMEM((SEC, CK), jnp.int32) for _ in range(2)],
            [pltpu.VMEM((SEC, CK), jnp.int32) for _ in range(2)],
            [pltpu.VMEM((SEC, CK), jnp.float32) for _ in range(2)],
            [pltpu.VMEM((CK, D), jnp.float32) for _ in range(NBUF)],
            pltpu.VMEM_SHARED((NN, D), jnp.float32),
            [pltpu.SemaphoreType.DMA for _ in range(2)],
            [pltpu.SemaphoreType.DMA for _ in range(NBUF)],
            [pltpu.SemaphoreType.DMA for _ in range(NBUF)],
        ],
    )
    def k(x_hbm, src_hbm, dst_hbm, w_hbm, part_hbm,
          src_s, dst_s, w_s, bufs, acc, isem, gsem, ssem):
        cid = lax.axis_index("c")
        sid = lax.axis_index("s")
        sec_base = sid * nsec2 + cid * N0
        nsec = jnp.where(cid == 0, N0, n1)
        nct = nsec * SEC
        nround = nsec // 2

        zbuf = bufs[0]

        def zrow(r, _):
            for c in range(CPD):
                zbuf[r, pl.ds(c * LANES, LANES)] = jnp.zeros((LANES,), jnp.float32)
            return 0
        lax.fori_loop(0, CK, zrow, 0)

        rpt = 624
        nfull = rpt // CK
        rem = rpt - nfull * CK
        tail_base = NS * rpt
        tail = NN - tail_base
        base = sid * rpt
        for t in range(nfull):
            pltpu.sync_copy(zbuf, acc.at[pl.ds(base + t * CK, CK)])
        if rem:
            pltpu.sync_copy(zbuf.at[pl.ds(0, rem)],
                            acc.at[pl.ds(base + nfull * CK, rem)])

        @pl.when(sid == NS - 1)
        def _():
            pltpu.sync_copy(zbuf.at[pl.ds(0, tail)],
                            acc.at[pl.ds(tail_base, tail)])
        plsc.subcore_barrier()

        def load_sec(s, p):
            b = sec_base + s
            pltpu.async_copy(src_hbm.at[b], src_s[p], isem[p])
            pltpu.async_copy(dst_hbm.at[b], dst_s[p], isem[p])
            pltpu.async_copy(w_hbm.at[b], w_s[p], isem[p])

        def wait_sec(p):
            pltpu.make_async_copy(src_hbm.at[0], src_s[p], isem[p]).wait()
            pltpu.make_async_copy(dst_hbm.at[0], dst_s[p], isem[p]).wait()
            pltpu.make_async_copy(w_hbm.at[0], w_s[p], isem[p]).wait()

        def start_gather(p, jj, b):
            pltpu.async_copy(x_hbm.at[src_s[p].at[jj]], bufs[b], gsem[b])

        load_sec(0, 0)
        wait_sec(0)
        load_sec(1, 1)
        start_gather(0, 0, 0)
        start_gather(0, 1, 1)

        def rnd(t, _):
            for p in range(2):
                s = 2 * t + p

                @pl.when(s + 1 < nsec)
                def _():
                    wait_sec(1 - p)
                for jj in range(SEC):
                    c = 2 * SEC * t + SEC * p + jj
                    b = (SEC * p + jj) % NBUF
                    b2 = (b + 2) % NBUF
                    pltpu.make_async_copy(x_hbm.at[src_s[p].at[jj]],
                                          bufs[b], gsem[b]).wait()

                    @pl.when(c >= 1)
                    def _():
                        pltpu.make_async_copy(bufs[b2],
                                              acc.at[dst_s[p].at[jj]],
                                              ssem[b2]).wait()

                    @pl.when(c + 2 < nct)
                    def _():
                        if jj == 0:
                            start_gather(p, 2, b2)
                        else:
                            start_gather(1 - p, jj - 1, b2)

                    def grp_body(g, _, _b=b, _p=p, _jj=jj):
                        wgrp = w_s[_p][_jj, pl.ds(g * LANES, LANES)]
                        for lane in range(LANES):
                            wv = wgrp[lane]
                            r = g * LANES + lane
                            for cc in range(CPD):
                                bufs[_b][r, pl.ds(cc * LANES, LANES)] = (
                                    bufs[_b][r, pl.ds(cc * LANES, LANES)] * wv)
                        return 0
                    lax.fori_loop(0, CK // LANES, grp_body, 0)
                    pltpu.async_copy(bufs[b], acc.at[dst_s[p].at[jj]],
                                     ssem[b], add=True)

                @pl.when(s + 2 < nsec)
                def _():
                    load_sec(s + 2, p)
            return 0
        lax.fori_loop(0, nround, rnd, 0)

        bl = SEC - 1
        pl_last = 1
        pltpu.make_async_copy(bufs[bl], acc.at[dst_s[pl_last].at[SEC - 1]],
                              ssem[bl]).wait()
        plsc.subcore_barrier()

        for t in range(nfull):
            pltpu.sync_copy(acc.at[pl.ds(base + t * CK, CK)],
                            part_hbm.at[cid, pl.ds(base + t * CK, CK)])
        if rem:
            pltpu.sync_copy(acc.at[pl.ds(base + nfull * CK, rem)],
                            part_hbm.at[cid, pl.ds(base + nfull * CK, rem)])

        @pl.when(sid == NS - 1)
        def _():
            pltpu.sync_copy(acc.at[pl.ds(tail_base, tail)],
                            part_hbm.at[cid, pl.ds(tail_base, tail)])

    return k(x, src4, dst4, w4)


def _sc_gather(light_tab, ego_tab, cat3):
    npc = cat3.shape[1]
    rows_pw = npc * CKE
    nrows = NW * rows_pw
    mesh = plsc.VectorSubcoreMesh(core_axis_name="c", subcore_axis_name="s")

    @functools.partial(
        pl.kernel,
        out_type=(
            jax.ShapeDtypeStruct((nrows, D), jnp.float32),
            jax.ShapeDtypeStruct((nrows, D), jnp.float32),
        ),
        mesh=mesh,
        scratch_types=[
            pltpu.VMEM((npc, CKE), jnp.int32),
            pltpu.VMEM((CKE, D), jnp.float32),
            pltpu.VMEM((CKE, D), jnp.float32),
            pltpu.SemaphoreType.DMA,
            pltpu.SemaphoreType.DMA,
        ],
    )
    def k(lt_hbm, et_hbm, idx_hbm, lo_hbm, eo_hbm, idx_v, bufa, bufb, sa, sb):
        cid = lax.axis_index("c")
        sid = lax.axis_index("s")
        wid = sid * NC + cid
        pltpu.sync_copy(idx_hbm.at[wid], idx_v)

        def chunk(j, _):
            ca = pltpu.async_copy(lt_hbm.at[idx_v.at[j]], bufa, sa)
            cb = pltpu.async_copy(et_hbm.at[idx_v.at[j]], bufb, sb)
            ca.wait()
            pltpu.sync_copy(bufa, lo_hbm.at[pl.ds(wid * rows_pw + j * CKE, CKE)])
            cb.wait()
            pltpu.sync_copy(bufb, eo_hbm.at[pl.ds(wid * rows_pw + j * CKE, CKE)])
            return 0
        lax.fori_loop(0, npc, chunk, 0)

    return k(light_tab, ego_tab, cat3)


def kernel(users, pos_items, neg_items, edge_index, edge_weight,
           user_sizes, item_sizes, user_table, item_table):
    ue = _masked(user_table, user_sizes)
    ie = _masked(item_table, item_sizes)
    x0 = jnp.concatenate([ue, ie], axis=0)

    src = edge_index[0].astype(jnp.int32)
    dst = edge_index[1].astype(jnp.int32)
    w = edge_weight.astype(jnp.float32)
    e = src.shape[0]
    unit = NW * SEC * CK * 2
    epad = ((e + unit - 1) // unit) * unit
    pad = epad - e
    nws = epad // (SEC * CK)
    src4 = jnp.pad(src, (0, pad)).reshape(nws, SEC, CK)
    dst4 = jnp.pad(dst, (0, pad)).reshape(nws, SEC, CK)
    w4 = jnp.pad(w, (0, pad)).reshape(nws, SEC, CK)

    x = x0
    s = x0
    for l in range(NLAYERS):
        part = _sc_layer(x, src4, dst4, w4)
        x, s = _combine(part, s, 0.25 if l == NLAYERS - 1 else 1.0)

    b = users.shape[0]
    cat = jnp.concatenate([
        users.astype(jnp.int32),
        pos_items.astype(jnp.int32) + NU,
        neg_items.astype(jnp.int32) + NU,
    ])
    cat3 = cat.reshape(NW, -1, CKE)
    light, ego = _sc_gather(s, x0, cat3)
    return (light[:b], light[b:2 * b], light[2 * b:],
            ego[:b], ego[b:2 * b], ego[2 * b:])
